# baseline (speedup 1.0000x reference)
import jax
import jax.numpy as jnp
from jax.experimental import pallas as pl
from jax.experimental.pallas import tpu as pltpu

H = 768
HD = H // 2
B, T = 8, 512
G4 = 4 * HD
BT = 64
NBLK = T // BT


def _dot(a, b):
    return jnp.dot(a, b, preferred_element_type=jnp.float32)


def _lstm_cell(gates, h, c, whh_ref):
    g = gates + _dot(h.astype(jnp.bfloat16), whh_ref[...])
    ig = jax.nn.sigmoid(g[:, 0:HD])
    fg = jax.nn.sigmoid(g[:, HD:2 * HD])
    gg = jnp.tanh(g[:, 2 * HD:3 * HD])
    og = jax.nn.sigmoid(g[:, 3 * HD:])
    c = fg * c + ig * gg
    h = og * jnp.tanh(c)
    return h, c


def _fused_kernel(xf_ref, xr_ref,
                  wihf0_ref, whhf0_ref, bf0_ref,
                  wihr0_ref, whhr0_ref, br0_ref,
                  w1fa_ref, w1fb_ref, whhf1_ref, bf1_ref,
                  w1ra_ref, w1rb_ref, whhr1_ref, br1_ref,
                  node_ref,
                  hsf_s, hsr_s,
                  hf_s, cf_s, hr_s, cr_s, accf_s, accr_s):
    i = pl.program_id(0)

    @pl.when((i == 0) | (i == NBLK))
    def _init():
        hf_s[...] = jnp.zeros_like(hf_s)
        cf_s[...] = jnp.zeros_like(cf_s)
        hr_s[...] = jnp.zeros_like(hr_s)
        cr_s[...] = jnp.zeros_like(cr_s)
        accf_s[...] = jnp.zeros_like(accf_s)
        accr_s[...] = jnp.zeros_like(accr_s)

    @pl.when(i < NBLK)
    def _layer0():
        xf = xf_ref[...].reshape(BT * B, H).astype(jnp.bfloat16)
        xr = xr_ref[...].reshape(BT * B, H).astype(jnp.bfloat16)
        gf = _dot(xf, wihf0_ref[...]) + bf0_ref[...]
        gr = _dot(xr, wihr0_ref[...]) + br0_ref[...]

        base_f = i * BT
        base_r = (NBLK - 1 - i) * BT
        hf, cf, hr, cr = hf_s[...], cf_s[...], hr_s[...], cr_s[...]
        for s in range(BT):
            sr = BT - 1 - s
            hf, cf = _lstm_cell(gf[s * B:(s + 1) * B], hf, cf, whhf0_ref)
            hsf_s[pl.ds(base_f + s, 1)] = hf.astype(jnp.bfloat16)[None]
            hr, cr = _lstm_cell(gr[sr * B:(sr + 1) * B], hr, cr, whhr0_ref)
            hsr_s[pl.ds(base_r + sr, 1)] = hr.astype(jnp.bfloat16)[None]
        hf_s[...], cf_s[...], hr_s[...], cr_s[...] = hf, cf, hr, cr

    @pl.when(i >= NBLK)
    def _layer1():
        j = i - NBLK
        base_f = j * BT
        base_r = (NBLK - 1 - j) * BT
        af = hsf_s[pl.ds(base_f, BT)].reshape(BT * B, HD)
        bf = hsr_s[pl.ds(base_f, BT)].reshape(BT * B, HD)
        ar = hsf_s[pl.ds(base_r, BT)].reshape(BT * B, HD)
        br = hsr_s[pl.ds(base_r, BT)].reshape(BT * B, HD)
        gf = _dot(af, w1fa_ref[...]) + _dot(bf, w1fb_ref[...]) + bf1_ref[...]
        gr = _dot(ar, w1ra_ref[...]) + _dot(br, w1rb_ref[...]) + br1_ref[...]

        hf, cf, hr, cr = hf_s[...], cf_s[...], hr_s[...], cr_s[...]
        accf, accr = accf_s[...], accr_s[...]
        for s in range(BT):
            sr = BT - 1 - s
            hf, cf = _lstm_cell(gf[s * B:(s + 1) * B], hf, cf, whhf1_ref)
            hr, cr = _lstm_cell(gr[sr * B:(sr + 1) * B], hr, cr, whhr1_ref)
            accf = accf + hf
            accr = accr + hr
        hf_s[...], cf_s[...], hr_s[...], cr_s[...] = hf, cf, hr, cr
        accf_s[...], accr_s[...] = accf, accr

    @pl.when(i == 2 * NBLK - 1)
    def _emit():
        inv_t = jnp.float32(1.0 / T)
        node_ref[:, 0:HD] = accf_s[...] * inv_t
        node_ref[:, HD:H] = accr_s[...] * inv_t


def _fwd_map(i):
    return (jnp.minimum(i, NBLK - 1), 0, 0)


def _rev_map(i):
    return (jnp.maximum(NBLK - 1 - i, 0), 0, 0)


def _full_map2(i):
    return (0, 0)


def kernel(context_hidden,
           W_ih_l0, W_hh_l0, b_ih_l0, b_hh_l0,
           W_ih_l0_r, W_hh_l0_r, b_ih_l0_r, b_hh_l0_r,
           W_ih_l1, W_hh_l1, b_ih_l1, b_hh_l1,
           W_ih_l1_r, W_hh_l1_r, b_ih_l1_r, b_hh_l1_r):
    f32 = jnp.float32
    bf16 = jnp.bfloat16
    x = jnp.swapaxes(context_hidden, 0, 1)

    def wspec(shape):
        return pl.BlockSpec(shape, _full_map2)

    wihf0 = W_ih_l0.T.astype(bf16)
    wihr0 = W_ih_l0_r.T.astype(bf16)
    whhf0 = W_hh_l0.T.astype(bf16)
    whhr0 = W_hh_l0_r.T.astype(bf16)
    bf0 = (b_ih_l0 + b_hh_l0).reshape(1, G4)
    br0 = (b_ih_l0_r + b_hh_l0_r).reshape(1, G4)
    wihf1 = W_ih_l1.T.astype(bf16)
    wihr1 = W_ih_l1_r.T.astype(bf16)
    whhf1 = W_hh_l1.T.astype(bf16)
    whhr1 = W_hh_l1_r.T.astype(bf16)
    bf1 = (b_ih_l1 + b_hh_l1).reshape(1, G4)
    br1 = (b_ih_l1_r + b_hh_l1_r).reshape(1, G4)

    node = pl.pallas_call(
        _fused_kernel,
        grid=(2 * NBLK,),
        in_specs=[pl.BlockSpec((BT, B, H), _fwd_map),
                  pl.BlockSpec((BT, B, H), _rev_map),
                  wspec((H, G4)), wspec((HD, G4)), wspec((1, G4)),
                  wspec((H, G4)), wspec((HD, G4)), wspec((1, G4)),
                  wspec((HD, G4)), wspec((HD, G4)), wspec((HD, G4)),
                  wspec((1, G4)),
                  wspec((HD, G4)), wspec((HD, G4)), wspec((HD, G4)),
                  wspec((1, G4))],
        out_specs=pl.BlockSpec((B, H), _full_map2),
        out_shape=jax.ShapeDtypeStruct((B, H), f32),
        scratch_shapes=[pltpu.VMEM((T, B, HD), jnp.bfloat16)] * 2
                       + [pltpu.VMEM((B, HD), f32)] * 6,
        compiler_params=pltpu.CompilerParams(
            dimension_semantics=("arbitrary",)),
    )(x, x, wihf0, whhf0, bf0, wihr0, whhr0, br0,
      wihf1[:HD], wihf1[HD:], whhf1, bf1,
      wihr1[:HD], wihr1[HD:], whhr1, br1)

    edge_index = jnp.array([[0, 1], [1, 0]], dtype=jnp.int32)
    edge_types = jnp.array([0, 0], dtype=jnp.int32)
    return node, edge_index, edge_types


# speedup vs baseline: 1.7258x; 1.7258x over previous
import jax
import jax.numpy as jnp
from jax.experimental import pallas as pl
from jax.experimental.pallas import tpu as pltpu

H = 768
HD = H // 2
B, T = 8, 512
G4 = 4 * HD
BT = 16
NBLK = T // BT


def _dot(a, b):
    return jnp.dot(a, b, preferred_element_type=jnp.float32)


def _lstm_cell(gates, h, c, whh_ref):
    g = gates + _dot(h.astype(jnp.bfloat16), whh_ref[...])
    ig = jax.nn.sigmoid(g[:, 0:HD])
    fg = jax.nn.sigmoid(g[:, HD:2 * HD])
    gg = jnp.tanh(g[:, 2 * HD:3 * HD])
    og = jax.nn.sigmoid(g[:, 3 * HD:])
    c = fg * c + ig * gg
    h = og * jnp.tanh(c)
    return h, c


def _fused_kernel(xf_ref, xr_ref,
                  wihf0_ref, whhf0_ref, bf0_ref,
                  wihr0_ref, whhr0_ref, br0_ref,
                  w1fa_ref, w1fb_ref, whhf1_ref, bf1_ref,
                  w1ra_ref, w1rb_ref, whhr1_ref, br1_ref,
                  node_ref,
                  hsf_s, hsr_s,
                  hf_s, cf_s, hr_s, cr_s, accf_s, accr_s):
    i = pl.program_id(0)

    @pl.when((i == 0) | (i == NBLK))
    def _init():
        hf_s[...] = jnp.zeros_like(hf_s)
        cf_s[...] = jnp.zeros_like(cf_s)
        hr_s[...] = jnp.zeros_like(hr_s)
        cr_s[...] = jnp.zeros_like(cr_s)
        accf_s[...] = jnp.zeros_like(accf_s)
        accr_s[...] = jnp.zeros_like(accr_s)

    @pl.when(i < NBLK)
    def _layer0():
        xf = xf_ref[...].reshape(BT * B, H).astype(jnp.bfloat16)
        xr = xr_ref[...].reshape(BT * B, H).astype(jnp.bfloat16)
        gf = _dot(xf, wihf0_ref[...]) + bf0_ref[...]
        gr = _dot(xr, wihr0_ref[...]) + br0_ref[...]

        base_f = i * BT
        base_r = (NBLK - 1 - i) * BT
        hf, cf, hr, cr = hf_s[...], cf_s[...], hr_s[...], cr_s[...]
        for s in range(BT):
            sr = BT - 1 - s
            hf, cf = _lstm_cell(gf[s * B:(s + 1) * B], hf, cf, whhf0_ref)
            hsf_s[pl.ds(base_f + s, 1)] = hf.astype(jnp.bfloat16)[None]
            hr, cr = _lstm_cell(gr[sr * B:(sr + 1) * B], hr, cr, whhr0_ref)
            hsr_s[pl.ds(base_r + sr, 1)] = hr.astype(jnp.bfloat16)[None]
        hf_s[...], cf_s[...], hr_s[...], cr_s[...] = hf, cf, hr, cr

    @pl.when(i >= NBLK)
    def _layer1():
        j = i - NBLK
        base_f = j * BT
        base_r = (NBLK - 1 - j) * BT
        af = hsf_s[pl.ds(base_f, BT)].reshape(BT * B, HD)
        bf = hsr_s[pl.ds(base_f, BT)].reshape(BT * B, HD)
        ar = hsf_s[pl.ds(base_r, BT)].reshape(BT * B, HD)
        br = hsr_s[pl.ds(base_r, BT)].reshape(BT * B, HD)
        gf = _dot(af, w1fa_ref[...]) + _dot(bf, w1fb_ref[...]) + bf1_ref[...]
        gr = _dot(ar, w1ra_ref[...]) + _dot(br, w1rb_ref[...]) + br1_ref[...]

        hf, cf, hr, cr = hf_s[...], cf_s[...], hr_s[...], cr_s[...]
        accf, accr = accf_s[...], accr_s[...]
        for s in range(BT):
            sr = BT - 1 - s
            hf, cf = _lstm_cell(gf[s * B:(s + 1) * B], hf, cf, whhf1_ref)
            hr, cr = _lstm_cell(gr[sr * B:(sr + 1) * B], hr, cr, whhr1_ref)
            accf = accf + hf
            accr = accr + hr
        hf_s[...], cf_s[...], hr_s[...], cr_s[...] = hf, cf, hr, cr
        accf_s[...], accr_s[...] = accf, accr

    @pl.when(i == 2 * NBLK - 1)
    def _emit():
        inv_t = jnp.float32(1.0 / T)
        node_ref[:, 0:HD] = accf_s[...] * inv_t
        node_ref[:, HD:H] = accr_s[...] * inv_t


def _fwd_map(i):
    return (jnp.minimum(i, NBLK - 1), 0, 0)


def _rev_map(i):
    return (jnp.maximum(NBLK - 1 - i, 0), 0, 0)


def _full_map2(i):
    return (0, 0)


def kernel(context_hidden,
           W_ih_l0, W_hh_l0, b_ih_l0, b_hh_l0,
           W_ih_l0_r, W_hh_l0_r, b_ih_l0_r, b_hh_l0_r,
           W_ih_l1, W_hh_l1, b_ih_l1, b_hh_l1,
           W_ih_l1_r, W_hh_l1_r, b_ih_l1_r, b_hh_l1_r):
    f32 = jnp.float32
    bf16 = jnp.bfloat16
    x = jnp.swapaxes(context_hidden, 0, 1)

    def wspec(shape):
        return pl.BlockSpec(shape, _full_map2)

    wihf0 = W_ih_l0.T.astype(bf16)
    wihr0 = W_ih_l0_r.T.astype(bf16)
    whhf0 = W_hh_l0.T.astype(bf16)
    whhr0 = W_hh_l0_r.T.astype(bf16)
    bf0 = (b_ih_l0 + b_hh_l0).reshape(1, G4)
    br0 = (b_ih_l0_r + b_hh_l0_r).reshape(1, G4)
    wihf1 = W_ih_l1.T.astype(bf16)
    wihr1 = W_ih_l1_r.T.astype(bf16)
    whhf1 = W_hh_l1.T.astype(bf16)
    whhr1 = W_hh_l1_r.T.astype(bf16)
    bf1 = (b_ih_l1 + b_hh_l1).reshape(1, G4)
    br1 = (b_ih_l1_r + b_hh_l1_r).reshape(1, G4)

    node = pl.pallas_call(
        _fused_kernel,
        grid=(2 * NBLK,),
        in_specs=[pl.BlockSpec((BT, B, H), _fwd_map),
                  pl.BlockSpec((BT, B, H), _rev_map),
                  wspec((H, G4)), wspec((HD, G4)), wspec((1, G4)),
                  wspec((H, G4)), wspec((HD, G4)), wspec((1, G4)),
                  wspec((HD, G4)), wspec((HD, G4)), wspec((HD, G4)),
                  wspec((1, G4)),
                  wspec((HD, G4)), wspec((HD, G4)), wspec((HD, G4)),
                  wspec((1, G4))],
        out_specs=pl.BlockSpec((B, H), _full_map2),
        out_shape=jax.ShapeDtypeStruct((B, H), f32),
        scratch_shapes=[pltpu.VMEM((T, B, HD), jnp.bfloat16)] * 2
                       + [pltpu.VMEM((B, HD), f32)] * 6,
        compiler_params=pltpu.CompilerParams(
            dimension_semantics=("arbitrary",)),
    )(x, x, wihf0, whhf0, bf0, wihr0, whhr0, br0,
      wihf1[:HD], wihf1[HD:], whhf1, bf1,
      wihr1[:HD], wihr1[HD:], whhr1, br1)

    edge_index = jnp.array([[0, 1], [1, 0]], dtype=jnp.int32)
    edge_types = jnp.array([0, 0], dtype=jnp.int32)
    return node, edge_index, edge_types


# speedup vs baseline: 1.8828x; 1.0910x over previous
import jax
import jax.numpy as jnp
from jax.experimental import pallas as pl
from jax.experimental.pallas import tpu as pltpu

H = 768
HD = H // 2
B, T = 8, 512
G4 = 4 * HD
BT = 32
NBLK = T // BT


def _dot(a, b):
    return jnp.dot(a, b, preferred_element_type=jnp.float32)


def _lstm_cell(gates, h, c, whh_ref):
    g = gates + _dot(h.astype(jnp.bfloat16), whh_ref[...])
    ig = jax.nn.sigmoid(g[:, 0:HD])
    fg = jax.nn.sigmoid(g[:, HD:2 * HD])
    gg = jnp.tanh(g[:, 2 * HD:3 * HD])
    og = jax.nn.sigmoid(g[:, 3 * HD:])
    c = fg * c + ig * gg
    h = og * jnp.tanh(c)
    return h, c


def _fused_kernel(xf_ref, xr_ref,
                  wihf0_ref, whhf0_ref, bf0_ref,
                  wihr0_ref, whhr0_ref, br0_ref,
                  w1fa_ref, w1fb_ref, whhf1_ref, bf1_ref,
                  w1ra_ref, w1rb_ref, whhr1_ref, br1_ref,
                  node_ref,
                  hsf_s, hsr_s,
                  hf_s, cf_s, hr_s, cr_s, accf_s, accr_s):
    i = pl.program_id(0)

    @pl.when((i == 0) | (i == NBLK))
    def _init():
        hf_s[...] = jnp.zeros_like(hf_s)
        cf_s[...] = jnp.zeros_like(cf_s)
        hr_s[...] = jnp.zeros_like(hr_s)
        cr_s[...] = jnp.zeros_like(cr_s)
        accf_s[...] = jnp.zeros_like(accf_s)
        accr_s[...] = jnp.zeros_like(accr_s)

    @pl.when(i < NBLK)
    def _layer0():
        xf = jnp.swapaxes(xf_ref[...], 0, 1).reshape(BT * B, H).astype(jnp.bfloat16)
        xr = jnp.swapaxes(xr_ref[...], 0, 1).reshape(BT * B, H).astype(jnp.bfloat16)
        gf = _dot(xf, wihf0_ref[...]) + bf0_ref[...]
        gr = _dot(xr, wihr0_ref[...]) + br0_ref[...]

        base_f = i * BT
        base_r = (NBLK - 1 - i) * BT
        hf, cf, hr, cr = hf_s[...], cf_s[...], hr_s[...], cr_s[...]
        for s in range(BT):
            sr = BT - 1 - s
            hf, cf = _lstm_cell(gf[s * B:(s + 1) * B], hf, cf, whhf0_ref)
            hsf_s[pl.ds(base_f + s, 1)] = hf.astype(jnp.bfloat16)[None]
            hr, cr = _lstm_cell(gr[sr * B:(sr + 1) * B], hr, cr, whhr0_ref)
            hsr_s[pl.ds(base_r + sr, 1)] = hr.astype(jnp.bfloat16)[None]
        hf_s[...], cf_s[...], hr_s[...], cr_s[...] = hf, cf, hr, cr

    @pl.when(i >= NBLK)
    def _layer1():
        j = i - NBLK
        base_f = j * BT
        base_r = (NBLK - 1 - j) * BT
        af = hsf_s[pl.ds(base_f, BT)].reshape(BT * B, HD)
        bf = hsr_s[pl.ds(base_f, BT)].reshape(BT * B, HD)
        ar = hsf_s[pl.ds(base_r, BT)].reshape(BT * B, HD)
        br = hsr_s[pl.ds(base_r, BT)].reshape(BT * B, HD)
        gf = _dot(af, w1fa_ref[...]) + _dot(bf, w1fb_ref[...]) + bf1_ref[...]
        gr = _dot(ar, w1ra_ref[...]) + _dot(br, w1rb_ref[...]) + br1_ref[...]

        hf, cf, hr, cr = hf_s[...], cf_s[...], hr_s[...], cr_s[...]
        accf, accr = accf_s[...], accr_s[...]
        for s in range(BT):
            sr = BT - 1 - s
            hf, cf = _lstm_cell(gf[s * B:(s + 1) * B], hf, cf, whhf1_ref)
            hr, cr = _lstm_cell(gr[sr * B:(sr + 1) * B], hr, cr, whhr1_ref)
            accf = accf + hf
            accr = accr + hr
        hf_s[...], cf_s[...], hr_s[...], cr_s[...] = hf, cf, hr, cr
        accf_s[...], accr_s[...] = accf, accr

    @pl.when(i == 2 * NBLK - 1)
    def _emit():
        inv_t = jnp.float32(1.0 / T)
        node_ref[:, 0:HD] = accf_s[...] * inv_t
        node_ref[:, HD:H] = accr_s[...] * inv_t


def _fwd_map_x(i):
    return (0, jnp.minimum(i, NBLK - 1), 0)


def _rev_map_x(i):
    return (0, jnp.maximum(NBLK - 1 - i, 0), 0)


def _full_map2(i):
    return (0, 0)


def kernel(context_hidden,
           W_ih_l0, W_hh_l0, b_ih_l0, b_hh_l0,
           W_ih_l0_r, W_hh_l0_r, b_ih_l0_r, b_hh_l0_r,
           W_ih_l1, W_hh_l1, b_ih_l1, b_hh_l1,
           W_ih_l1_r, W_hh_l1_r, b_ih_l1_r, b_hh_l1_r):
    f32 = jnp.float32
    bf16 = jnp.bfloat16

    def wspec(shape):
        return pl.BlockSpec(shape, _full_map2)

    wihf0 = W_ih_l0.T.astype(bf16)
    wihr0 = W_ih_l0_r.T.astype(bf16)
    whhf0 = W_hh_l0.T.astype(bf16)
    whhr0 = W_hh_l0_r.T.astype(bf16)
    bf0 = (b_ih_l0 + b_hh_l0).reshape(1, G4)
    br0 = (b_ih_l0_r + b_hh_l0_r).reshape(1, G4)
    wihf1 = W_ih_l1.T.astype(bf16)
    wihr1 = W_ih_l1_r.T.astype(bf16)
    whhf1 = W_hh_l1.T.astype(bf16)
    whhr1 = W_hh_l1_r.T.astype(bf16)
    bf1 = (b_ih_l1 + b_hh_l1).reshape(1, G4)
    br1 = (b_ih_l1_r + b_hh_l1_r).reshape(1, G4)

    node = pl.pallas_call(
        _fused_kernel,
        grid=(2 * NBLK,),
        in_specs=[pl.BlockSpec((B, BT, H), _fwd_map_x),
                  pl.BlockSpec((B, BT, H), _rev_map_x),
                  wspec((H, G4)), wspec((HD, G4)), wspec((1, G4)),
                  wspec((H, G4)), wspec((HD, G4)), wspec((1, G4)),
                  wspec((HD, G4)), wspec((HD, G4)), wspec((HD, G4)),
                  wspec((1, G4)),
                  wspec((HD, G4)), wspec((HD, G4)), wspec((HD, G4)),
                  wspec((1, G4))],
        out_specs=pl.BlockSpec((B, H), _full_map2),
        out_shape=jax.ShapeDtypeStruct((B, H), f32),
        scratch_shapes=[pltpu.VMEM((T, B, HD), jnp.bfloat16)] * 2
                       + [pltpu.VMEM((B, HD), f32)] * 6,
        compiler_params=pltpu.CompilerParams(
            dimension_semantics=("arbitrary",)),
    )(context_hidden, context_hidden, wihf0, whhf0, bf0, wihr0, whhr0, br0,
      wihf1[:HD], wihf1[HD:], whhf1, bf1,
      wihr1[:HD], wihr1[HD:], whhr1, br1)

    edge_index = jnp.array([[0, 1], [1, 0]], dtype=jnp.int32)
    edge_types = jnp.array([0, 0], dtype=jnp.int32)
    return node, edge_index, edge_types
